# baseline (speedup 1.0000x reference)
import functools

import jax
import jax.numpy as jnp
from jax import lax
from jax.experimental import pallas as pl
from jax.experimental.pallas import tpu as pltpu
from jax.experimental.pallas import tpu_sc as plsc

SCALE = 8.0

IW = 128
IB = 8


@functools.lru_cache(maxsize=None)
def _build(B, V, D):
    info = plsc.get_sparse_core_info()
    nw = info.num_cores * info.num_subcores
    rows_w = (B // IW) // nw
    nb = rows_w // IB
    blk = IB * IW

    @functools.partial(
        pl.kernel,
        out_type=jax.ShapeDtypeStruct((B, D), jnp.float32),
        mesh=plsc.VectorSubcoreMesh(core_axis_name="c", subcore_axis_name="s"),
        compiler_params=pltpu.CompilerParams(use_tc_tiling_on_sc=False),
        scratch_types=[
            pltpu.VMEM((IB, IW), jnp.int32),
            pltpu.VMEM((blk, D), jnp.float32),
            pltpu.SemaphoreType.DMA,
        ],
    )
    def emb(tok_hbm, table_hbm, out_hbm, idx_v, rows_v, sem):
        wid = lax.axis_index("s") * info.num_cores + lax.axis_index("c")
        row0 = wid * rows_w

        def block(b, carry):
            r = row0 + b * IB
            pltpu.sync_copy(tok_hbm.at[pl.ds(r, IB)], idx_v)
            cps = [
                pltpu.async_copy(
                    table_hbm.at[idx_v.at[j]],
                    rows_v.at[pl.ds(j * IW, IW)],
                    sem,
                )
                for j in range(IB)
            ]
            for cp in cps:
                cp.wait()

            def srow(i, c2):
                row = rows_v.at[i]
                for j in range(D // 16):
                    row[pl.ds(j * 16, 16)] = row[pl.ds(j * 16, 16)] * SCALE
                return c2

            lax.fori_loop(0, blk, srow, 0)
            pltpu.sync_copy(rows_v, out_hbm.at[pl.ds(r * IW, blk)])
            return carry

        lax.fori_loop(0, nb, block, 0)

    return emb


def kernel(tokens, table):
    B = tokens.size
    V, D = table.shape
    tok2d = tokens.reshape(B // IW, IW).astype(jnp.int32)
    out = _build(B, V, D)(tok2d, table)
    return out.reshape(*tokens.shape, D)
